# baseline (speedup 1.0000x reference)
import math

import jax
import jax.numpy as jnp
import numpy as np
from jax import lax
from jax.experimental import pallas as pl
from jax.experimental.pallas import tpu as pltpu
from jax.experimental.pallas import tpu_sc as plsc

N = 2048
D = 768
U = 160
NCLS = 16


def _threefry2x32(k1, k2, x0, x1):
    rots = ((13, 15, 26, 6), (17, 29, 16, 24))
    ks = (np.uint32(k1), np.uint32(k2),
          np.uint32(k1) ^ np.uint32(k2) ^ np.uint32(0x1BD11BDA))
    x0 = x0 + ks[0]
    x1 = x1 + ks[1]
    for i in range(5):
        for r in rots[i % 2]:
            x0 = x0 + x1
            x1 = (x1 << np.uint32(r)) | (x1 >> np.uint32(32 - r))
            x1 = x0 ^ x1
        x0 = x0 + ks[(i + 1) % 3]
        x1 = x1 + ks[(i + 2) % 3] + np.uint32(i + 1)
    return x0, x1


def _index_sample_constant():
    old = np.seterr(over="ignore")
    try:
        sk_hi, sk_lo = _threefry2x32(
            0, 42, np.zeros(2, np.uint32), np.arange(2, dtype=np.uint32))
        k1, k2 = sk_hi[1], sk_lo[1]
        size = N * U
        hb, lb = _threefry2x32(
            k1, k2, np.zeros(size, np.uint32), np.arange(size, dtype=np.uint32))
        bits = hb ^ lb
        return (bits % np.uint32(N)).astype(np.int64).reshape(N, U)
    finally:
        np.seterr(**old)


_idx = _index_sample_constant()
_cnt_np = np.zeros((N, N), np.uint8)
np.add.at(_cnt_np, (np.arange(N)[:, None], _idx), 1)
_CNT = _cnt_np

_UT128 = np.triu(np.ones((128, 128), np.float32))
_LT16S = np.tril(np.ones((16, 16), np.float32), k=-1)

_QB = 256

_NW = 32
_SC_OFF = (N * D) // 2
_WPT = _SC_OFF // _NW
_CH = 3072
_NCHUNK = _WPT // _CH
_TCCH = 98304


def _proj_body(x_ref, wq_ref, wk_ref, wv_ref, wa_ref,
               q_ref, k_ref, v_ref, a_ref):
    x = x_ref[...]
    dn = (((1,), (1,)), ((), ()))
    q_ref[...] = jax.lax.dot_general(x, wq_ref[...], dn,
                                     preferred_element_type=jnp.float32)
    k_ref[...] = jax.lax.dot_general(x, wk_ref[...], dn,
                                     preferred_element_type=jnp.float32)
    v_ref[...] = jax.lax.dot_general(x, wv_ref[...], dn,
                                     preferred_element_type=jnp.float32)
    a_ref[...] = jax.lax.dot_general(x, wa_ref[...], dn,
                                     preferred_element_type=jnp.float32)


def _vmean_body(v_ref, o_ref):
    o_ref[...] = jnp.mean(v_ref[...], axis=0, keepdims=True)


def _attn_body(q_ref, k_ref, v_ref, cnt_ref, m_ref, ctx_ref):
    q = q_ref[...]
    k = k_ref[...]
    s = jax.lax.dot_general(q, k, (((1,), (1,)), ((), ())),
                            preferred_element_type=jnp.float32)
    cnt = cnt_ref[...].astype(jnp.float32)
    mmax = jnp.max(jnp.where(cnt > 0.0, s, -jnp.inf), axis=1)
    msum = jnp.sum(s * cnt, axis=1)
    m_ref[...] = (mmax - msum * (1.0 / N)).reshape(1, 1, _QB)
    ss = s * (1.0 / math.sqrt(D))
    rm = jnp.max(ss, axis=1, keepdims=True)
    e = jnp.exp(ss - rm)
    p = e / jnp.sum(e, axis=1, keepdims=True)
    ctx_ref[...] = jnp.dot(p, v_ref[...], preferred_element_type=jnp.float32)


def _sel_body(m_ref, ut_ref, lt_ref, mtop_ref):
    m = m_ref[...]
    u = jax.lax.bitcast_convert_type(m, jnp.int32)
    key = jnp.where(u < 0, u ^ jnp.int32(0x7FFFFFFF), u)
    row = jax.lax.broadcasted_iota(jnp.int32, (16, 128), 0)
    col = jax.lax.broadcasted_iota(jnp.int32, (16, 128), 1)
    idx = row * 128 + col

    def t_body(b, t):
        tp = t + jnp.left_shift(jnp.int32(1), 31 - b)
        c = jnp.sum((key >= tp).astype(jnp.int32))
        return jnp.where(c >= U, tp, t)

    t = jax.lax.fori_loop(0, 32, t_body, jnp.int32(-2147483647 - 1))
    gt = key > t
    eq = key == t
    need = U - jnp.sum(gt.astype(jnp.int32))

    def j_body(b, j):
        jp = j + jnp.left_shift(jnp.int32(1), 10 - b)
        c = jnp.sum((eq & (idx <= jp)).astype(jnp.int32))
        return jnp.where(c <= need, jp, j)

    j = jax.lax.fori_loop(0, 11, j_body, jnp.int32(-1))
    sel = gt | (eq & (idx <= j))
    self32 = sel.astype(jnp.float32)

    pos_in = jnp.dot(self32, ut_ref[...],
                     preferred_element_type=jnp.float32)
    row_tot = jnp.sum(self32, axis=1, keepdims=True)
    row_off = jnp.dot(lt_ref[...], row_tot,
                      preferred_element_type=jnp.float32)
    pos = (pos_in + row_off) * self32

    jpl = jax.lax.broadcasted_iota(jnp.int32, (U, 16, 128), 0).astype(jnp.float32)
    hit = (pos[None, :, :] == (jpl + 1.0)) & sel[None, :, :]
    idxf = idx.astype(jnp.float32)
    mtopf = jnp.sum(jnp.where(hit, idxf[None, :, :], 0.0), axis=(1, 2))
    mtop_ref[...] = mtopf.astype(jnp.int32).reshape(1, U)


def _corr_body(mtop_ref, w_ref, ctx_ref, vm_ref, o_ref):
    row = ctx_ref[pl.ds(mtop_ref[pl.program_id(0)], 1), :]
    contrib = jnp.sum(w_ref[...] * (row - vm_ref[...]), axis=1)

    @pl.when(pl.program_id(0) == 0)
    def _():
        o_ref[...] = jnp.zeros_like(o_ref)

    o_ref[...] += contrib.reshape(1, NCLS)


def _tcfinal_body(ctxf_ref, w_ref, o_ref):
    i = pl.program_id(0)
    crow = ctxf_ref[pl.ds(i, 1), :]
    w = w_ref[...]
    contrib = jnp.sum(w * crow, axis=1)

    @pl.when(i == 0)
    def _():
        o_ref[...] = jnp.zeros_like(o_ref)

    o_ref[...] += contrib.reshape(1, NCLS)


def _sc_base_body(w_hbm, ctx_hbm, out_hbm, wbuf, cbuf, obuf,
                  wsem0, wsem1, csem0, csem1):
    wid = lax.axis_index("s") * 2 + lax.axis_index("c")
    base = wid * _WPT
    wsems = (wsem0, wsem1)
    csems = (csem0, csem1)

    def launch(kk):
        slot = kk % 2
        wcp = pltpu.make_async_copy(
            w_hbm.at[:, pl.ds(_SC_OFF + base + kk * _CH, _CH)], wbuf.at[slot],
            wsems[slot])
        wcp.start()
        ccp = pltpu.make_async_copy(
            ctx_hbm.at[pl.ds(base + kk * _CH, _CH)], cbuf.at[slot],
            csems[slot])
        ccp.start()
        return wcp, ccp

    accs = tuple(jnp.zeros((16,), jnp.float32) for _ in range(NCLS))
    pending = launch(0)
    for kk in range(_NCHUNK):
        slot = kk % 2
        nxt = launch(kk + 1) if kk + 1 < _NCHUNK else None
        pending[0].wait()
        pending[1].wait()

        def inner(r, acc):
            cv = cbuf[slot, pl.ds(r * 16, 16)]
            return tuple(acc[c] + wbuf[slot, c, pl.ds(r * 16, 16)] * cv
                         for c in range(NCLS))

        accs = lax.fori_loop(0, _CH // 16, inner, accs)
        pending = nxt
    for c in range(NCLS):
        obuf[c, :] = accs[c]
    pltpu.sync_copy(obuf, out_hbm.at[wid])


def kernel(input_embedding, fai_x, fai_x_prime, w_1, b_1, w_2, b_2,
           Wq, Wk, Wv, Wadd, badd, Wfin, bfin):
    x = input_embedding.reshape(N, D)

    q, k, v, add = pl.pallas_call(
        _proj_body,
        grid=(N // _QB,),
        in_specs=[
            pl.BlockSpec((_QB, D), lambda i: (i, 0)),
            pl.BlockSpec((D, D), lambda i: (0, 0)),
            pl.BlockSpec((D, D), lambda i: (0, 0)),
            pl.BlockSpec((D, D), lambda i: (0, 0)),
            pl.BlockSpec((D, D), lambda i: (0, 0)),
        ],
        out_specs=[pl.BlockSpec((_QB, D), lambda i: (i, 0))] * 4,
        out_shape=[jax.ShapeDtypeStruct((N, D), jnp.float32)] * 4,
    )(x, Wq, Wk, Wv, Wadd)

    vmean = pl.pallas_call(
        _vmean_body,
        out_shape=jax.ShapeDtypeStruct((1, D), jnp.float32),
    )(v)

    base = add + vmean + badd[None, :]
    base_lo = base[:N // 2].reshape(8, _TCCH)
    base_hi = base[N // 2:].reshape(_SC_OFF)

    sc_base = pl.kernel(
        _sc_base_body,
        mesh=plsc.VectorSubcoreMesh(core_axis_name="c", subcore_axis_name="s"),
        out_type=jax.ShapeDtypeStruct((_NW, NCLS, 16), jnp.float32),
        scratch_types=[
            pltpu.VMEM((2, NCLS, _CH), jnp.float32),
            pltpu.VMEM((2, _CH), jnp.float32),
            pltpu.VMEM((NCLS, 16), jnp.float32),
            pltpu.SemaphoreType.DMA,
            pltpu.SemaphoreType.DMA,
            pltpu.SemaphoreType.DMA,
            pltpu.SemaphoreType.DMA,
        ],
    )
    partials = sc_base(Wfin, base_hi)

    out_lo = pl.pallas_call(
        _tcfinal_body,
        grid=(8,),
        in_specs=[
            pl.BlockSpec((8, _TCCH), lambda i: (0, 0)),
            pl.BlockSpec((NCLS, _TCCH), lambda i: (0, i)),
        ],
        out_specs=pl.BlockSpec((1, NCLS), lambda i: (0, 0)),
        out_shape=jax.ShapeDtypeStruct((1, NCLS), jnp.float32),
    )(base_lo, Wfin)

    m3, ctx_attn = pl.pallas_call(
        _attn_body,
        grid=(N // _QB,),
        in_specs=[
            pl.BlockSpec((_QB, D), lambda i: (i, 0)),
            pl.BlockSpec((N, D), lambda i: (0, 0)),
            pl.BlockSpec((N, D), lambda i: (0, 0)),
            pl.BlockSpec((_QB, N), lambda i: (i, 0)),
        ],
        out_specs=[
            pl.BlockSpec((1, 1, _QB), lambda i: (i, 0, 0)),
            pl.BlockSpec((_QB, D), lambda i: (i, 0)),
        ],
        out_shape=[
            jax.ShapeDtypeStruct((N // _QB, 1, _QB), jnp.float32),
            jax.ShapeDtypeStruct((N, D), jnp.float32),
        ],
    )(q, k, v, _CNT)

    mtop = pl.pallas_call(
        _sel_body,
        out_shape=jax.ShapeDtypeStruct((1, U), jnp.int32),
    )(m3.reshape(16, 128), _UT128, _LT16S)

    corr = pl.pallas_call(
        _corr_body,
        grid_spec=pltpu.PrefetchScalarGridSpec(
            num_scalar_prefetch=1,
            grid=(U,),
            in_specs=[
                pl.BlockSpec((NCLS, D), lambda i, mt: (0, mt[i])),
                pl.BlockSpec((N, D), lambda i, mt: (0, 0)),
                pl.BlockSpec((1, D), lambda i, mt: (0, 0)),
            ],
            out_specs=pl.BlockSpec((1, NCLS), lambda i, mt: (0, 0)),
        ),
        out_shape=jax.ShapeDtypeStruct((1, NCLS), jnp.float32),
    )(mtop.reshape(U), Wfin, ctx_attn, vmean)

    out_base = jnp.sum(partials, axis=(0, 2))
    return out_lo + out_base[None, :] + corr + bfin[None, :]


# speedup vs baseline: 1.7416x; 1.7416x over previous
import math

import jax
import jax.numpy as jnp
import numpy as np
from jax.experimental import pallas as pl

N = 2048
D = 768
U = 160
NCLS = 16


def _threefry2x32(k1, k2, x0, x1):
    rots = ((13, 15, 26, 6), (17, 29, 16, 24))
    ks = (np.uint32(k1), np.uint32(k2),
          np.uint32(k1) ^ np.uint32(k2) ^ np.uint32(0x1BD11BDA))
    x0 = x0 + ks[0]
    x1 = x1 + ks[1]
    for i in range(5):
        for r in rots[i % 2]:
            x0 = x0 + x1
            x1 = (x1 << np.uint32(r)) | (x1 >> np.uint32(32 - r))
            x1 = x0 ^ x1
        x0 = x0 + ks[(i + 1) % 3]
        x1 = x1 + ks[(i + 2) % 3] + np.uint32(i + 1)
    return x0, x1


def _index_sample_constant():
    old = np.seterr(over="ignore")
    try:
        sk_hi, sk_lo = _threefry2x32(
            0, 42, np.zeros(2, np.uint32), np.arange(2, dtype=np.uint32))
        k1, k2 = sk_hi[1], sk_lo[1]
        size = N * U
        hb, lb = _threefry2x32(
            k1, k2, np.zeros(size, np.uint32), np.arange(size, dtype=np.uint32))
        bits = hb ^ lb
        return (bits % np.uint32(N)).astype(np.int64).reshape(N, U)
    finally:
        np.seterr(**old)


_idx = _index_sample_constant()
_cnt_np = np.zeros((N, N), np.uint8)
np.add.at(_cnt_np, (np.arange(N)[:, None], _idx), 1)
_CNT = _cnt_np

_QB = 256
_FB = 128


def _proj_body(x_ref, wq_ref, wk_ref, wv_ref, wa_ref,
               q_ref, k_ref, v_ref, a_ref):
    x = x_ref[...]
    dn = (((1,), (1,)), ((), ()))
    q_ref[...] = jax.lax.dot_general(x, wq_ref[...], dn,
                                     preferred_element_type=jnp.float32)
    k_ref[...] = jax.lax.dot_general(x, wk_ref[...], dn,
                                     preferred_element_type=jnp.float32)
    v_ref[...] = jax.lax.dot_general(x, wv_ref[...], dn,
                                     preferred_element_type=jnp.float32)
    a_ref[...] = jax.lax.dot_general(x, wa_ref[...], dn,
                                     preferred_element_type=jnp.float32)


def _attn_body(q_ref, k_ref, v_ref, cnt_ref, m_ref, ctx_ref, vm_ref):
    q = q_ref[...]
    k = k_ref[...]

    @pl.when(pl.program_id(0) == 0)
    def _():
        vm_ref[...] = jnp.mean(v_ref[...], axis=0, keepdims=True)
    s = jax.lax.dot_general(q, k, (((1,), (1,)), ((), ())),
                            preferred_element_type=jnp.float32)
    cnt = cnt_ref[...].astype(jnp.float32)
    mmax = jnp.max(jnp.where(cnt > 0.0, s, -jnp.inf), axis=1)
    msum = jnp.sum(s * cnt, axis=1)
    m_ref[...] = (mmax - msum * (1.0 / N)).reshape(1, 1, _QB)
    ss = s * (1.0 / math.sqrt(D))
    rm = jnp.max(ss, axis=1, keepdims=True)
    e = jnp.exp(ss - rm)
    p = e / jnp.sum(e, axis=1, keepdims=True)
    ctx_ref[...] = jnp.dot(p, v_ref[...], preferred_element_type=jnp.float32)


def _sel_body(m_ref, sel_ref):
    m = m_ref[...]
    u = jax.lax.bitcast_convert_type(m, jnp.int32)
    key = jnp.where(u < 0, u ^ jnp.int32(0x7FFFFFFF), u)
    row = jax.lax.broadcasted_iota(jnp.int32, (16, 128), 0)
    col = jax.lax.broadcasted_iota(jnp.int32, (16, 128), 1)
    idx = row * 128 + col

    def t_body(b, t):
        tp = t + jnp.left_shift(jnp.int32(1), 31 - b)
        c = jnp.sum((key >= tp).astype(jnp.int32))
        return jnp.where(c >= U, tp, t)

    t = jax.lax.fori_loop(0, 32, t_body, jnp.int32(-2147483647 - 1))
    gt = key > t
    eq = key == t
    need = U - jnp.sum(gt.astype(jnp.int32))

    def j_body(b, j):
        jp = j + jnp.left_shift(jnp.int32(1), 10 - b)
        c = jnp.sum((eq & (idx <= jp)).astype(jnp.int32))
        return jnp.where(c <= need, jp, j)

    j = jax.lax.fori_loop(0, 11, j_body, jnp.int32(-1))
    sel = gt | (eq & (idx <= j))
    sel_ref[...] = sel.astype(jnp.float32)


def _blend_body(ctx_ref, add_ref, sel_ref, vm_ref, badd_ref, o_ref):
    sel = sel_ref[:, 0:1]
    ctx = ctx_ref[...] * sel + vm_ref[...] * (1.0 - sel)
    o_ref[...] = ctx + add_ref[...] + badd_ref[...]


def _final_body(ctxf_ref, w_ref, o_ref):
    i = pl.program_id(0)
    crow = ctxf_ref[pl.ds(i, 1), :]
    w = w_ref[...]
    contrib = jnp.sum(w * crow, axis=1)

    @pl.when(i == 0)
    def _():
        o_ref[...] = jnp.zeros_like(o_ref)

    o_ref[...] += contrib.reshape(1, NCLS)


def kernel(input_embedding, fai_x, fai_x_prime, w_1, b_1, w_2, b_2,
           Wq, Wk, Wv, Wadd, badd, Wfin, bfin):
    x = input_embedding.reshape(N, D)

    q, k, v, add = pl.pallas_call(
        _proj_body,
        grid=(N // _QB,),
        in_specs=[
            pl.BlockSpec((_QB, D), lambda i: (i, 0)),
            pl.BlockSpec((D, D), lambda i: (0, 0)),
            pl.BlockSpec((D, D), lambda i: (0, 0)),
            pl.BlockSpec((D, D), lambda i: (0, 0)),
            pl.BlockSpec((D, D), lambda i: (0, 0)),
        ],
        out_specs=[pl.BlockSpec((_QB, D), lambda i: (i, 0))] * 4,
        out_shape=[jax.ShapeDtypeStruct((N, D), jnp.float32)] * 4,
    )(x, Wq, Wk, Wv, Wadd)

    m3, ctx_attn, vmean = pl.pallas_call(
        _attn_body,
        grid=(N // _QB,),
        in_specs=[
            pl.BlockSpec((_QB, D), lambda i: (i, 0)),
            pl.BlockSpec((N, D), lambda i: (0, 0)),
            pl.BlockSpec((N, D), lambda i: (0, 0)),
            pl.BlockSpec((_QB, N), lambda i: (i, 0)),
        ],
        out_specs=[
            pl.BlockSpec((1, 1, _QB), lambda i: (i, 0, 0)),
            pl.BlockSpec((_QB, D), lambda i: (i, 0)),
            pl.BlockSpec((1, D), lambda i: (0, 0)),
        ],
        out_shape=[
            jax.ShapeDtypeStruct((N // _QB, 1, _QB), jnp.float32),
            jax.ShapeDtypeStruct((N, D), jnp.float32),
            jax.ShapeDtypeStruct((1, D), jnp.float32),
        ],
    )(q, k, v, _CNT)

    sel = pl.pallas_call(
        _sel_body,
        out_shape=jax.ShapeDtypeStruct((16, 128), jnp.float32),
    )(m3.reshape(16, 128))

    selb = jnp.broadcast_to(sel.reshape(N, 1), (N, 128))

    ctx_final = pl.pallas_call(
        _blend_body,
        grid=(N // _FB,),
        in_specs=[
            pl.BlockSpec((_FB, D), lambda i: (i, 0)),
            pl.BlockSpec((_FB, D), lambda i: (i, 0)),
            pl.BlockSpec((_FB, 128), lambda i: (i, 0)),
            pl.BlockSpec((1, D), lambda i: (0, 0)),
            pl.BlockSpec((1, D), lambda i: (0, 0)),
        ],
        out_specs=pl.BlockSpec((_FB, D), lambda i: (i, 0)),
        out_shape=jax.ShapeDtypeStruct((N, D), jnp.float32),
    )(ctx_attn, add, selb, vmean, badd.reshape(1, D))

    chunk = N * D // 8
    ctxf = ctx_final.reshape(8, chunk)

    out = pl.pallas_call(
        _final_body,
        grid=(8,),
        in_specs=[
            pl.BlockSpec((8, chunk), lambda i: (0, 0)),
            pl.BlockSpec((NCLS, chunk), lambda i: (0, i)),
        ],
        out_specs=pl.BlockSpec((1, NCLS), lambda i: (0, 0)),
        out_shape=jax.ShapeDtypeStruct((1, NCLS), jnp.float32),
    )(ctxf, Wfin)

    return out + bfin[None, :]
